# Initial kernel scaffold; baseline (speedup 1.0000x reference)
import jax
import jax.numpy as jnp
from jax.experimental import pallas as pl


def kernel(queries, keys, values, key_lengths_additive, planes):
    raise NotImplementedError("write your pallas kernel here")


# speedup vs baseline: 3.6136x; 3.6136x over previous
import functools

import jax
import jax.numpy as jnp
import numpy as np
from jax.experimental import pallas as pl

CLUSTERS = 128
ITERATIONS = 10
BITS = 32


def _attn_body(q_ref, k_ref, v_ref, kadd_ref, planes_ref, out_ref):
    L, E = q_ref.shape[1], q_ref.shape[2]
    C = CLUSTERS
    q = q_ref[0]
    w = planes_ref[:, :E]
    b = planes_ref[:, E]

    proj = jnp.dot(q, w.T, preferred_element_type=jnp.float32) + b[None, :]
    hb = (proj > 0.0).astype(jnp.float32)

    shifts = jax.lax.broadcasted_iota(jnp.int32, (L, BITS), 1)
    hpacked = jnp.sum(hb.astype(jnp.int32) << shifts, axis=-1,
                      keepdims=True)

    cpacked0 = hpacked.reshape(C, L // C)[:, :1].reshape(1, C)

    iota_c = jax.lax.broadcasted_iota(jnp.int32, (L, C), 1)
    cshifts = jax.lax.broadcasted_iota(jnp.int32, (C, BITS), 1)

    def lloyd(_, carry):
        cpacked, _, _ = carry
        dist = jax.lax.population_count(hpacked ^ cpacked)
        key = (dist << 7) | iota_c
        kmin = jnp.min(key, axis=-1, keepdims=True)
        onehot = (key == kmin).astype(jnp.float32)
        counts = jnp.sum(onehot, axis=0)
        bitsum = jnp.dot(onehot.T, hb, preferred_element_type=jnp.float32)
        newcb = (bitsum / jnp.maximum(counts, 1.0)[:, None] > 0.5)
        newcp = jnp.sum(newcb.astype(jnp.int32) << cshifts, axis=-1)
        cpacked = jnp.where((counts > 0.0)[None, :], newcp[None, :], cpacked)
        return cpacked, onehot, counts

    init = (cpacked0, jnp.zeros((L, C), jnp.float32),
            jnp.zeros((C,), jnp.float32))
    _, onehot, counts = jax.lax.fori_loop(0, ITERATIONS, lloyd, init)

    factors = 1.0 / jnp.maximum(counts, 1.0)
    q_grouped = jnp.dot(onehot.T, q, preferred_element_type=jnp.float32)
    q_grouped = q_grouped * factors[:, None]

    k = k_ref[0]
    qk = jax.lax.dot_general(
        q_grouped, k, (((1,), (1,)), ((), ())),
        preferred_element_type=jnp.float32)
    qk = qk + kadd_ref[0, 0][None, :]
    temp = 1.0 / np.sqrt(E).astype(np.float32)
    a = jax.nn.softmax(temp * qk, axis=-1)
    v_grouped = jnp.dot(a, v_ref[0], preferred_element_type=jnp.float32)

    out_ref[0] = jnp.dot(onehot, v_grouped, preferred_element_type=jnp.float32)


@jax.jit
def kernel(queries, keys, values, key_lengths_additive, planes):
    N, L, H, E = queries.shape
    NH = N * H
    q = jnp.transpose(queries, (0, 2, 1, 3)).reshape(NH, L, E)
    k = jnp.transpose(keys, (0, 2, 1, 3)).reshape(NH, L, E)
    v = jnp.transpose(values, (0, 2, 1, 3)).reshape(NH, L, E)
    kadd = key_lengths_additive.reshape(N, 1, L)

    out = pl.pallas_call(
        _attn_body,
        grid=(NH,),
        in_specs=[
            pl.BlockSpec((1, L, E), lambda i: (i, 0, 0)),
            pl.BlockSpec((1, L, E), lambda i: (i, 0, 0)),
            pl.BlockSpec((1, L, E), lambda i: (i, 0, 0)),
            pl.BlockSpec((1, 1, L), lambda i: (i // H, 0, 0)),
            pl.BlockSpec((BITS, E + 1), lambda i: (0, 0)),
        ],
        out_specs=pl.BlockSpec((1, L, E), lambda i: (i, 0, 0)),
        out_shape=jax.ShapeDtypeStruct((NH, L, E), jnp.float32),
    )(q, k, v, kadd, planes)

    return jnp.transpose(out.reshape(N, H, L, E), (0, 2, 1, 3))
